# baseline (speedup 1.0000x reference)
import jax, jax.numpy as jnp
import math

EF = 32


def _square_distance(src, dst):
    d = -2.0 * jnp.matmul(src, jnp.swapaxes(dst, 1, 2))
    d = d + jnp.sum(src ** 2, -1)[:, :, None]
    d = d + jnp.sum(dst ** 2, -1)[:, None, :]
    return d


def _index_points(points, idx):
    Bb = points.shape[0]
    batch = jnp.arange(Bb).reshape((Bb,) + (1,) * (idx.ndim - 1))
    return points[batch, idx]


def _knn_point(nsample, xyz, new_xyz):
    sq = _square_distance(new_xyz, xyz)
    neg, idx = jax.lax.top_k(-sq, nsample)
    return idx, -neg


def _fps(xyz, npoint):
    Bb, Nn, _ = xyz.shape
    dist = jnp.full((Bb, Nn), 1e10, dtype=jnp.float32)
    farthest = jnp.zeros((Bb,), dtype=jnp.int32)
    ar = jnp.arange(Bb)
    idxs = []
    for i in range(npoint):
        idxs.append(farthest)
        centroid = xyz[ar, farthest][:, None, :]
        d = jnp.sum((xyz - centroid) ** 2, -1)
        dist = jnp.minimum(dist, d)
        farthest = jnp.argmax(dist, -1).astype(jnp.int32)
    return jnp.stack(idxs, axis=1)


def _transformer_apply(p, feature, xyz, out_dim, knn_num=36):
    Bb, Nn, _ = feature.shape
    point_index, _ = _knn_point(knn_num, xyz, xyz)
    pre_weight = jnp.concatenate([feature, xyz], -1)
    g_weight = _index_points(pre_weight, point_index) - pre_weight[:, :, None, :]
    rep = jnp.broadcast_to(pre_weight[:, :, None, :], (Bb, Nn, knn_num, pre_weight.shape[-1]))
    g_weight = jnp.concatenate([g_weight, rep], -1)
    weight = (jnp.matmul(g_weight, p["r_W"]) + p["r_b"]).reshape(Bb, Nn, -1, out_dim)
    weight_abs = jnp.abs(weight) + 1e-07
    weight = weight / jnp.sum(weight_abs, -1, keepdims=True) * math.sqrt(out_dim)
    gv = jax.nn.relu(jnp.matmul(pre_weight, p["v_W"]) + p["v_b"])
    group_feature = _index_points(gv, point_index)
    feat = jnp.matmul(group_feature.reshape(Bb, Nn, 1, -1), weight)[:, :, 0, :]
    feat = jnp.matmul(feat, p["s_W"]) + p["s_b"]
    return feat


def _bn_apply(p, x, eps=1e-05):
    mean = jnp.mean(x, axis=(0, 1), keepdims=True)
    var = jnp.var(x, axis=(0, 1), keepdims=True)
    return (x - mean) / jnp.sqrt(var + eps) * p["gamma"] + p["beta"]


def _indicator_apply(p, feature1, xyz1, xyz2, knn_num=12):
    feature1 = jnp.matmul(feature1, p["pre_W"]) + p["pre_b"]
    point_index, distance = _knn_point(knn_num, xyz1, xyz2)
    min_distance = distance[:, :, 0]
    weight = jnp.where(min_distance > 0.03, 10.0, 1.0)
    g_feature = _index_points(feature1, point_index)
    g_xyz = _index_points(xyz1, point_index) - xyz2[:, :, None, :]
    h = jax.nn.relu(jnp.matmul(g_xyz, p["p1_W"]) + p["p1_b"])
    position_weight = jnp.matmul(h, p["p2_W"]) + p["p2_b"]
    new_feature = jnp.sum(position_weight * g_feature, axis=2) / math.sqrt(knn_num)
    return new_feature, weight


def kernel(xyz, detect_point, normal_gt, params):
    f1 = _transformer_apply(params["tl1"], xyz, xyz, EF // 4)
    f1 = _bn_apply(params["bn1"], f1)
    f2 = _transformer_apply(params["tl2"], f1, xyz, EF)
    f2 = _bn_apply(params["bn2"], f2)
    f3 = _transformer_apply(params["tl3"], f2, xyz, EF)
    f3 = _bn_apply(params["bn3"], f3)
    far_idx = _fps(xyz, 512)
    f3_512 = _index_points(f3, far_idx)
    xyz512 = _index_points(xyz, far_idx)
    f4 = _transformer_apply(params["tl4"], f3_512, xyz512, EF)
    f4 = _bn_apply(params["bn4"], f4)
    f4_up, _ = _indicator_apply(params["i5"], f4, xyz512, xyz, knn_num=12)
    f5 = _transformer_apply(params["tl5"], f4_up, xyz, EF)
    f5 = _bn_apply(params["bn5"], f5)
    feature = jnp.concatenate([f3, f5], -1)
    nf3, weight = _indicator_apply(params["i3"], feature, xyz, detect_point, knn_num=12)
    h = jax.nn.relu(jnp.matmul(nf3, params["cls"]["c1_W"]) + params["cls"]["c1_b"])
    occ = jnp.matmul(h, params["cls"]["c2_W"]) + params["cls"]["c2_b"]
    return occ, weight


# speedup vs baseline: 1.7130x; 1.7130x over previous
import functools
import math

import jax
import jax.numpy as jnp
from jax import lax
from jax.experimental import pallas as pl
from jax.experimental.pallas import tpu as pltpu

_HIGH = lax.Precision.HIGHEST
_INTERP = False

EFD = 32
_B, _N, _M = 2, 1024, 2048


def _dot(a, b):
    return lax.dot_general(a, b, (((1,), (0,)), ((), ())), precision=_HIGH)


def _dot_nt(a, b):
    return lax.dot_general(a, b, (((1,), (1,)), ((), ())), precision=_HIGH)


def _knn_body(k, q_ref, c_ref, idx_ref, w_ref):
    q = q_ref[0]
    c = c_ref[0]
    qb = q.shape[0]
    cn = c.shape[0]
    qaug = jnp.concatenate([-2.0 * q, jnp.ones((qb, 1), jnp.float32)], axis=1)
    csq = jnp.sum(c * c, axis=1, keepdims=True)
    caug = jnp.concatenate([c, csq], axis=1)
    s = _dot_nt(qaug, caug) + jnp.sum(q * q, axis=1, keepdims=True)
    iota = lax.broadcasted_iota(jnp.int32, (qb, cn), 1)
    cols = []
    for j in range(k):
        m = jnp.min(s, axis=1, keepdims=True)
        if j == 0:
            w_ref[0] = jnp.where(m > 0.03, 10.0, 1.0)
        sel = jnp.where(s <= m, iota, cn)
        ij = jnp.min(sel, axis=1, keepdims=True)
        cols.append(ij)
        s = jnp.where(iota == ij, jnp.float32(3.0e38), s)
    idx_ref[0] = jnp.concatenate(cols, axis=1)


def _knn(query, cand, k, qblk=256):
    bb, qn, _ = query.shape
    cn = cand.shape[1]
    idx, w = pl.pallas_call(
        functools.partial(_knn_body, k),
        grid=(bb, qn // qblk),
        in_specs=[pl.BlockSpec((1, qblk, 3), lambda b, i: (b, i, 0)),
                  pl.BlockSpec((1, cn, 3), lambda b, i: (b, 0, 0))],
        out_specs=(pl.BlockSpec((1, qblk, k), lambda b, i: (b, i, 0)),
                   pl.BlockSpec((1, qblk, 1), lambda b, i: (b, i, 0))),
        out_shape=(jax.ShapeDtypeStruct((bb, qn, k), jnp.int32),
                   jax.ShapeDtypeStruct((bb, qn, 1), jnp.float32)),
        interpret=_INTERP,
    )(query, cand)
    return idx, w


def _fps_body(npoint, xyz_ref, f3_ref, of_ref, ox_ref, oh_ref):
    xyz = xyz_ref[0]
    n = xyz.shape[0]
    riota = lax.broadcasted_iota(jnp.int32, (n, 1), 0)
    liota = lax.broadcasted_iota(jnp.int32, (1, n), 1)

    def body(i, carry):
        dist, far = carry
        oh_ref[pl.ds(i, 1), :] = (liota == far).astype(jnp.float32)
        cen = xyz_ref[0, pl.ds(far, 1), :]
        d = jnp.sum((xyz - cen) ** 2, axis=1, keepdims=True)
        dist = jnp.minimum(dist, d)
        m = jnp.max(dist)
        far2 = jnp.min(jnp.where(dist >= m, riota, n)).astype(jnp.int32)
        return dist, far2

    dist0 = jnp.full((n, 1), 1e10, jnp.float32)
    lax.fori_loop(0, npoint, body, (dist0, jnp.int32(0)))
    oh = oh_ref[:, :]
    of_ref[0] = _dot(oh, f3_ref[0])
    ox_ref[0] = _dot(oh, xyz)


def _fps_gather(xyz, f3, npoint=512):
    bb, n, _ = xyz.shape
    cf = f3.shape[-1]
    f3s, x512 = pl.pallas_call(
        functools.partial(_fps_body, npoint),
        grid=(bb,),
        in_specs=[pl.BlockSpec((1, n, 3), lambda b: (b, 0, 0)),
                  pl.BlockSpec((1, n, cf), lambda b: (b, 0, 0))],
        out_specs=(pl.BlockSpec((1, npoint, cf), lambda b: (b, 0, 0)),
                   pl.BlockSpec((1, npoint, 3), lambda b: (b, 0, 0))),
        out_shape=(jax.ShapeDtypeStruct((bb, npoint, cf), jnp.float32),
                   jax.ShapeDtypeStruct((bb, npoint, 3), jnp.float32)),
        scratch_shapes=[pltpu.VMEM((npoint, n), jnp.float32)],
        interpret=_INTERP,
    )(xyz, f3)
    return f3s, x512


def _tl_body(knn, dd, dout, pb, pre_ref, idx_ref, rw1_ref, rw2_ref, rb_ref,
             vw_ref, vb_ref, sw_ref, sb_ref, sum_ref, exp_ref, p_ref,
             out_ref):
    i = pl.program_id(1)
    pre = pre_ref[0]
    n = pre.shape[0]
    gv = jnp.maximum(_dot(pre, vw_ref[...]) + vb_ref[...], 0.0)
    tab = jnp.concatenate([gv, pre], axis=1)
    q = pre_ref[0, pl.ds(i * pb, pb), :]
    qw2 = _dot(q, rw2_ref[...]) + rb_ref[...]
    idxb = idx_ref[0]
    liota = lax.broadcasted_iota(jnp.int32, (pb, n), 1)
    rw1 = rw1_ref[...]
    smat = sum_ref[...]
    emat = exp_ref[...]
    pmat = p_ref[...]
    rootd = math.sqrt(dout)
    feat = jnp.zeros((pb, dout), jnp.float32)
    for j in range(knn):
        oh = (idxb[:, j:j + 1] == liota).astype(jnp.float32)
        nbr = _dot(oh, tab)
        gvn = nbr[:, :dout]
        dn = nbr[:, dout:] - q
        mj = _dot(dn, rw1) + qw2
        sj = _dot(jnp.abs(mj), smat) + jnp.float32(dout * 1e-7)
        a = gvn * rootd / sj
        a_exp = _dot(a, emat)
        feat = feat + _dot(a_exp * mj, pmat)
    out_ref[0] = _dot(feat, sw_ref[...]) + sb_ref[...]


def _tlayer(pre, idx, p, dout, pblk=256):
    bb, n, dd = pre.shape
    knn = idx.shape[-1]
    d2 = dout * dout
    rw1 = p["r_W"][:dd]
    rw2 = p["r_W"][dd:]
    rb = p["r_b"].reshape(1, d2)
    vw = p["v_W"]
    vb = p["v_b"].reshape(1, dout)
    sw = p["s_W"]
    sb = p["s_b"].reshape(1, dout)
    eye = jnp.eye(dout, dtype=jnp.float32)
    ones = jnp.ones((dout, 1), jnp.float32)
    smat = jnp.kron(eye, ones)
    pmat = jnp.kron(ones, eye)
    emat = smat.T
    wspec = lambda w: pl.BlockSpec(w.shape, lambda b, i: (0,) * w.ndim)
    out = pl.pallas_call(
        functools.partial(_tl_body, knn, dd, dout, pblk),
        grid=(bb, n // pblk),
        in_specs=[pl.BlockSpec((1, n, dd), lambda b, i: (b, 0, 0)),
                  pl.BlockSpec((1, pblk, knn), lambda b, i: (b, i, 0)),
                  wspec(rw1), wspec(rw2), wspec(rb), wspec(vw), wspec(vb),
                  wspec(sw), wspec(sb), wspec(smat), wspec(emat),
                  wspec(pmat)],
        out_specs=pl.BlockSpec((1, pblk, dout), lambda b, i: (b, i, 0)),
        out_shape=jax.ShapeDtypeStruct((bb, n, dout), jnp.float32),
        interpret=_INTERP,
    )(pre, idx, rw1, rw2, rb, vw, vb, sw, sb, smat, emat, pmat)
    return out


def _bn_body(x_ref, g_ref, b_ref, o_ref):
    x = x_ref[...]
    m = jnp.mean(x, axis=(0, 1), keepdims=True)
    v = jnp.mean((x - m) ** 2, axis=(0, 1), keepdims=True)
    o_ref[...] = (x - m) / jnp.sqrt(v + 1e-5) * g_ref[...] + b_ref[...]


def _bn(x, p):
    bb, n, c = x.shape
    g = p["gamma"].reshape(1, 1, c)
    b = p["beta"].reshape(1, 1, c)
    return pl.pallas_call(
        _bn_body,
        in_specs=[pl.BlockSpec(x.shape, lambda: (0, 0, 0)),
                  pl.BlockSpec(g.shape, lambda: (0, 0, 0)),
                  pl.BlockSpec(b.shape, lambda: (0, 0, 0))],
        out_specs=pl.BlockSpec(x.shape, lambda: (0, 0, 0)),
        out_shape=jax.ShapeDtypeStruct(x.shape, jnp.float32),
        interpret=_INTERP,
    )(x, g, b)


def _ind_body(knn, dout, pb, with_cls, f1_ref, x1_ref, x2_ref, idx_ref,
              prew_ref, preb_ref, p1w_ref, p1b_ref, p2w_ref, p2b_ref,
              c1w_ref, c1b_ref, c2w_ref, c2b_ref, out_ref):
    i = pl.program_id(1)
    f1 = f1_ref[0]
    x1 = x1_ref[0]
    n1 = x1.shape[0]
    ft = _dot(f1, prew_ref[...]) + preb_ref[...]
    tab = jnp.concatenate([ft, x1], axis=1)
    q = x2_ref[0, pl.ds(i * pb, pb), :]
    idxb = idx_ref[0]
    liota = lax.broadcasted_iota(jnp.int32, (pb, n1), 1)
    p1w = p1w_ref[...]
    p1b = p1b_ref[...]
    p2w = p2w_ref[...]
    p2b = p2b_ref[...]
    acc = jnp.zeros((pb, dout), jnp.float32)
    for j in range(knn):
        oh = (idxb[:, j:j + 1] == liota).astype(jnp.float32)
        nbr = _dot(oh, tab)
        gx = nbr[:, dout:] - q
        h = jnp.maximum(_dot(gx, p1w) + p1b, 0.0)
        pw = _dot(h, p2w) + p2b
        acc = acc + pw * nbr[:, :dout]
    nf = acc / math.sqrt(knn)
    if with_cls:
        h2 = jnp.maximum(_dot(nf, c1w_ref[...]) + c1b_ref[...], 0.0)
        out_ref[0] = _dot(h2, c2w_ref[...]) + c2b_ref[...]
    else:
        out_ref[0] = nf


def _indicator(p, feat1, xyz1, xyz2, idx, cls=None, pblk=256):
    bb, n1, din = feat1.shape
    qn = xyz2.shape[1]
    knn = idx.shape[-1]
    dout = p["pre_W"].shape[1]
    prew = p["pre_W"]
    preb = p["pre_b"].reshape(1, dout)
    p1w = p["p1_W"]
    p1b = p["p1_b"].reshape(1, dout)
    p2w = p["p2_W"]
    p2b = p["p2_b"].reshape(1, dout)
    if cls is not None:
        c1w, c1b = cls["c1_W"], cls["c1_b"].reshape(1, -1)
        c2w, c2b = cls["c2_W"], cls["c2_b"].reshape(1, -1)
        cout = c2w.shape[1]
    else:
        c1w = c1b = c2w = c2b = jnp.zeros((1, 1), jnp.float32)
        cout = dout
    wspec = lambda w: pl.BlockSpec(w.shape, lambda b, i: (0,) * w.ndim)
    out = pl.pallas_call(
        functools.partial(_ind_body, knn, dout, pblk, cls is not None),
        grid=(bb, qn // pblk),
        in_specs=[pl.BlockSpec((1, n1, din), lambda b, i: (b, 0, 0)),
                  pl.BlockSpec((1, n1, 3), lambda b, i: (b, 0, 0)),
                  pl.BlockSpec((1, qn, 3), lambda b, i: (b, 0, 0)),
                  pl.BlockSpec((1, pblk, knn), lambda b, i: (b, i, 0)),
                  wspec(prew), wspec(preb), wspec(p1w), wspec(p1b),
                  wspec(p2w), wspec(p2b), wspec(c1w), wspec(c1b),
                  wspec(c2w), wspec(c2b)],
        out_specs=pl.BlockSpec((1, pblk, cout), lambda b, i: (b, i, 0)),
        out_shape=jax.ShapeDtypeStruct((bb, qn, cout), jnp.float32),
        interpret=_INTERP,
    )(feat1, xyz1, xyz2, idx, prew, preb, p1w, p1b, p2w, p2b,
      c1w, c1b, c2w, c2b)
    return out


def kernel(xyz, detect_point, normal_gt, params):
    del normal_gt
    idx36, _ = _knn(xyz, xyz, 36)

    pre1 = jnp.concatenate([xyz, xyz], axis=-1)
    f1 = _bn(_tlayer(pre1, idx36, params["tl1"], EFD // 4), params["bn1"])
    pre2 = jnp.concatenate([f1, xyz], axis=-1)
    f2 = _bn(_tlayer(pre2, idx36, params["tl2"], EFD), params["bn2"])
    pre3 = jnp.concatenate([f2, xyz], axis=-1)
    f3 = _bn(_tlayer(pre3, idx36, params["tl3"], EFD), params["bn3"])

    f3_512, xyz512 = _fps_gather(xyz, f3)

    idx36b, _ = _knn(xyz512, xyz512, 36)
    pre4 = jnp.concatenate([f3_512, xyz512], axis=-1)
    f4 = _bn(_tlayer(pre4, idx36b, params["tl4"], EFD), params["bn4"])

    idx12a, _ = _knn(xyz, xyz512, 12)
    f4_up = _indicator(params["i5"], f4, xyz512, xyz, idx12a)

    pre5 = jnp.concatenate([f4_up, xyz], axis=-1)
    f5 = _bn(_tlayer(pre5, idx36, params["tl5"], EFD), params["bn5"])

    idx12b, w = _knn(detect_point, xyz, 12)
    featc = jnp.concatenate([f3, f5], axis=-1)
    occ = _indicator(params["i3"], featc, xyz, detect_point, idx12b,
                     cls=params["cls"])
    return occ, w.reshape(w.shape[0], w.shape[1])


# speedup vs baseline: 1.8065x; 1.0546x over previous
import functools
import math

import jax
import jax.numpy as jnp
from jax import lax
from jax.experimental import pallas as pl
from jax.experimental.pallas import tpu as pltpu

_HIGH = lax.Precision.HIGHEST
_INTERP = False

EFD = 32
_B, _N, _M = 2, 1024, 2048


def _dot(a, b):
    return lax.dot_general(a, b, (((1,), (0,)), ((), ())), precision=_HIGH)


def _dot_nt(a, b):
    return lax.dot_general(a, b, (((1,), (1,)), ((), ())), precision=_HIGH)


def _knn_body(k, q_ref, c_ref, idx_ref, w_ref):
    q = q_ref[0]
    c = c_ref[0]
    qb = q.shape[0]
    cn = c.shape[0]
    qaug = jnp.concatenate([-2.0 * q, jnp.ones((qb, 1), jnp.float32)], axis=1)
    csq = jnp.sum(c * c, axis=1, keepdims=True)
    caug = jnp.concatenate([c, csq], axis=1)
    s = _dot_nt(qaug, caug) + jnp.sum(q * q, axis=1, keepdims=True)
    iota = lax.broadcasted_iota(jnp.int32, (qb, cn), 1)
    cols = []
    for j in range(k):
        m = jnp.min(s, axis=1, keepdims=True)
        if j == 0:
            w_ref[0] = jnp.where(m > 0.03, 10.0, 1.0)
        sel = jnp.where(s <= m, iota, cn)
        ij = jnp.min(sel, axis=1, keepdims=True)
        cols.append(ij)
        s = jnp.where(iota == ij, jnp.float32(3.0e38), s)
    idx_ref[0] = jnp.concatenate(cols, axis=1)


def _knn(query, cand, k, qblk=256):
    bb, qn, _ = query.shape
    cn = cand.shape[1]
    idx, w = pl.pallas_call(
        functools.partial(_knn_body, k),
        grid=(bb, qn // qblk),
        in_specs=[pl.BlockSpec((1, qblk, 3), lambda b, i: (b, i, 0)),
                  pl.BlockSpec((1, cn, 3), lambda b, i: (b, 0, 0))],
        out_specs=(pl.BlockSpec((1, qblk, k), lambda b, i: (b, i, 0)),
                   pl.BlockSpec((1, qblk, 1), lambda b, i: (b, i, 0))),
        out_shape=(jax.ShapeDtypeStruct((bb, qn, k), jnp.int32),
                   jax.ShapeDtypeStruct((bb, qn, 1), jnp.float32)),
        interpret=_INTERP,
    )(query, cand)
    return idx, w


def _fps_body(npoint, xyz_ref, f3_ref, of_ref, ox_ref, oh_ref):
    xyz = xyz_ref[0]
    n = xyz.shape[0]
    riota = lax.broadcasted_iota(jnp.int32, (n, 1), 0)
    liota = lax.broadcasted_iota(jnp.int32, (1, n), 1)

    def body(i, carry):
        dist, far = carry
        oh_ref[pl.ds(i, 1), :] = (liota == far).astype(jnp.float32)
        cen = xyz_ref[0, pl.ds(far, 1), :]
        d = jnp.sum((xyz - cen) ** 2, axis=1, keepdims=True)
        dist = jnp.minimum(dist, d)
        m = jnp.max(dist)
        far2 = jnp.min(jnp.where(dist >= m, riota, n)).astype(jnp.int32)
        return dist, far2

    dist0 = jnp.full((n, 1), 1e10, jnp.float32)
    lax.fori_loop(0, npoint, body, (dist0, jnp.int32(0)))
    oh = oh_ref[:, :]
    of_ref[0] = _dot(oh, f3_ref[0])
    ox_ref[0] = _dot(oh, xyz)


def _fps_gather(xyz, f3, npoint=512):
    bb, n, _ = xyz.shape
    cf = f3.shape[-1]
    f3s, x512 = pl.pallas_call(
        functools.partial(_fps_body, npoint),
        grid=(bb,),
        in_specs=[pl.BlockSpec((1, n, 3), lambda b: (b, 0, 0)),
                  pl.BlockSpec((1, n, cf), lambda b: (b, 0, 0))],
        out_specs=(pl.BlockSpec((1, npoint, cf), lambda b: (b, 0, 0)),
                   pl.BlockSpec((1, npoint, 3), lambda b: (b, 0, 0))),
        out_shape=(jax.ShapeDtypeStruct((bb, npoint, cf), jnp.float32),
                   jax.ShapeDtypeStruct((bb, npoint, 3), jnp.float32)),
        scratch_shapes=[pltpu.VMEM((npoint, n), jnp.float32)],
        interpret=_INTERP,
    )(xyz, f3)
    return f3s, x512


def _tl_body(knn, jc, dout, pb, pre_ref, idx_ref, rw1_ref, rw2_ref, rb_ref,
             vw_ref, vb_ref, sw_ref, sb_ref, sum_ref, exp_ref, p_ref,
             out_ref):
    i = pl.program_id(1)
    pre = pre_ref[0]
    n = pre.shape[0]
    gv = jnp.maximum(_dot(pre, vw_ref[...]) + vb_ref[...], 0.0)
    tab = jnp.concatenate([gv, pre], axis=1)
    q = pre_ref[0, pl.ds(i * pb, pb), :]
    rw1 = rw1_ref[...]
    rwc = rw2_ref[...] - rw1
    qcomb = _dot(q, rwc) + rb_ref[...]
    qrep = jnp.concatenate([qcomb] * jc, axis=0)
    idxb = idx_ref[0]
    liota = lax.broadcasted_iota(jnp.int32, (pb, n), 1)
    smat = sum_ref[...]
    emat = exp_ref[...]
    pmat = p_ref[...]
    rootd = math.sqrt(dout)
    feat = jnp.zeros((pb, dout), jnp.float32)
    for t in range(knn // jc):
        oh = jnp.concatenate(
            [(idxb[:, j:j + 1] == liota).astype(jnp.float32)
             for j in range(t * jc, (t + 1) * jc)], axis=0)
        nbr = _dot(oh, tab)
        gvn = nbr[:, :dout]
        mj = _dot(nbr[:, dout:], rw1) + qrep
        sj = _dot(jnp.abs(mj), smat) + jnp.float32(dout * 1e-7)
        a = gvn * rootd / sj
        a_exp = _dot(a, emat)
        contrib = _dot(a_exp * mj, pmat)
        feat = feat + jnp.sum(contrib.reshape(jc, pb, dout), axis=0)
    out_ref[0] = _dot(feat, sw_ref[...]) + sb_ref[...]


def _tlayer(pre, idx, p, dout, pblk=256, jc=6):
    bb, n, dd = pre.shape
    knn = idx.shape[-1]
    d2 = dout * dout
    rw1 = p["r_W"][:dd]
    rw2 = p["r_W"][dd:]
    rb = p["r_b"].reshape(1, d2)
    vw = p["v_W"]
    vb = p["v_b"].reshape(1, dout)
    sw = p["s_W"]
    sb = p["s_b"].reshape(1, dout)
    eye = jnp.eye(dout, dtype=jnp.float32)
    ones = jnp.ones((dout, 1), jnp.float32)
    smat = jnp.kron(eye, ones)
    pmat = jnp.kron(ones, eye)
    emat = smat.T
    wspec = lambda w: pl.BlockSpec(w.shape, lambda b, i: (0,) * w.ndim)
    out = pl.pallas_call(
        functools.partial(_tl_body, knn, jc, dout, pblk),
        grid=(bb, n // pblk),
        in_specs=[pl.BlockSpec((1, n, dd), lambda b, i: (b, 0, 0)),
                  pl.BlockSpec((1, pblk, knn), lambda b, i: (b, i, 0)),
                  wspec(rw1), wspec(rw2), wspec(rb), wspec(vw), wspec(vb),
                  wspec(sw), wspec(sb), wspec(smat), wspec(emat),
                  wspec(pmat)],
        out_specs=pl.BlockSpec((1, pblk, dout), lambda b, i: (b, i, 0)),
        out_shape=jax.ShapeDtypeStruct((bb, n, dout), jnp.float32),
        interpret=_INTERP,
    )(pre, idx, rw1, rw2, rb, vw, vb, sw, sb, smat, emat, pmat)
    return out


def _bn_body(x_ref, g_ref, b_ref, o_ref):
    x = x_ref[...]
    m = jnp.mean(x, axis=(0, 1), keepdims=True)
    v = jnp.mean((x - m) ** 2, axis=(0, 1), keepdims=True)
    o_ref[...] = (x - m) / jnp.sqrt(v + 1e-5) * g_ref[...] + b_ref[...]


def _bn(x, p):
    bb, n, c = x.shape
    g = p["gamma"].reshape(1, 1, c)
    b = p["beta"].reshape(1, 1, c)
    return pl.pallas_call(
        _bn_body,
        in_specs=[pl.BlockSpec(x.shape, lambda: (0, 0, 0)),
                  pl.BlockSpec(g.shape, lambda: (0, 0, 0)),
                  pl.BlockSpec(b.shape, lambda: (0, 0, 0))],
        out_specs=pl.BlockSpec(x.shape, lambda: (0, 0, 0)),
        out_shape=jax.ShapeDtypeStruct(x.shape, jnp.float32),
        interpret=_INTERP,
    )(x, g, b)


def _ind_body(knn, jc, dout, pb, with_cls, f1_ref, x1_ref, x2_ref, idx_ref,
              prew_ref, preb_ref, p1w_ref, p1b_ref, p2w_ref, p2b_ref,
              c1w_ref, c1b_ref, c2w_ref, c2b_ref, out_ref):
    i = pl.program_id(1)
    f1 = f1_ref[0]
    x1 = x1_ref[0]
    n1 = x1.shape[0]
    ft = _dot(f1, prew_ref[...]) + preb_ref[...]
    tab = jnp.concatenate([ft, x1], axis=1)
    q = x2_ref[0, pl.ds(i * pb, pb), :]
    qrep = jnp.concatenate([q] * jc, axis=0)
    idxb = idx_ref[0]
    liota = lax.broadcasted_iota(jnp.int32, (pb, n1), 1)
    p1w = p1w_ref[...]
    p1b = p1b_ref[...]
    p2w = p2w_ref[...]
    p2b = p2b_ref[...]
    acc = jnp.zeros((pb, dout), jnp.float32)
    for t in range(knn // jc):
        oh = jnp.concatenate(
            [(idxb[:, j:j + 1] == liota).astype(jnp.float32)
             for j in range(t * jc, (t + 1) * jc)], axis=0)
        nbr = _dot(oh, tab)
        gx = nbr[:, dout:] - qrep
        h = jnp.maximum(_dot(gx, p1w) + p1b, 0.0)
        pw = _dot(h, p2w) + p2b
        c = pw * nbr[:, :dout]
        acc = acc + jnp.sum(c.reshape(jc, pb, dout), axis=0)
    nf = acc / math.sqrt(knn)
    if with_cls:
        h2 = jnp.maximum(_dot(nf, c1w_ref[...]) + c1b_ref[...], 0.0)
        out_ref[0] = _dot(h2, c2w_ref[...]) + c2b_ref[...]
    else:
        out_ref[0] = nf


def _indicator(p, feat1, xyz1, xyz2, idx, cls=None, pblk=256, jc=6):
    bb, n1, din = feat1.shape
    qn = xyz2.shape[1]
    knn = idx.shape[-1]
    dout = p["pre_W"].shape[1]
    prew = p["pre_W"]
    preb = p["pre_b"].reshape(1, dout)
    p1w = p["p1_W"]
    p1b = p["p1_b"].reshape(1, dout)
    p2w = p["p2_W"]
    p2b = p["p2_b"].reshape(1, dout)
    if cls is not None:
        c1w, c1b = cls["c1_W"], cls["c1_b"].reshape(1, -1)
        c2w, c2b = cls["c2_W"], cls["c2_b"].reshape(1, -1)
        cout = c2w.shape[1]
    else:
        c1w = c1b = c2w = c2b = jnp.zeros((1, 1), jnp.float32)
        cout = dout
    wspec = lambda w: pl.BlockSpec(w.shape, lambda b, i: (0,) * w.ndim)
    out = pl.pallas_call(
        functools.partial(_ind_body, knn, jc, dout, pblk, cls is not None),
        grid=(bb, qn // pblk),
        in_specs=[pl.BlockSpec((1, n1, din), lambda b, i: (b, 0, 0)),
                  pl.BlockSpec((1, n1, 3), lambda b, i: (b, 0, 0)),
                  pl.BlockSpec((1, qn, 3), lambda b, i: (b, 0, 0)),
                  pl.BlockSpec((1, pblk, knn), lambda b, i: (b, i, 0)),
                  wspec(prew), wspec(preb), wspec(p1w), wspec(p1b),
                  wspec(p2w), wspec(p2b), wspec(c1w), wspec(c1b),
                  wspec(c2w), wspec(c2b)],
        out_specs=pl.BlockSpec((1, pblk, cout), lambda b, i: (b, i, 0)),
        out_shape=jax.ShapeDtypeStruct((bb, qn, cout), jnp.float32),
        interpret=_INTERP,
    )(feat1, xyz1, xyz2, idx, prew, preb, p1w, p1b, p2w, p2b,
      c1w, c1b, c2w, c2b)
    return out


def kernel(xyz, detect_point, normal_gt, params):
    del normal_gt
    idx36, _ = _knn(xyz, xyz, 36)

    pre1 = jnp.concatenate([xyz, xyz], axis=-1)
    f1 = _bn(_tlayer(pre1, idx36, params["tl1"], EFD // 4), params["bn1"])
    pre2 = jnp.concatenate([f1, xyz], axis=-1)
    f2 = _bn(_tlayer(pre2, idx36, params["tl2"], EFD), params["bn2"])
    pre3 = jnp.concatenate([f2, xyz], axis=-1)
    f3 = _bn(_tlayer(pre3, idx36, params["tl3"], EFD), params["bn3"])

    f3_512, xyz512 = _fps_gather(xyz, f3)

    idx36b, _ = _knn(xyz512, xyz512, 36)
    pre4 = jnp.concatenate([f3_512, xyz512], axis=-1)
    f4 = _bn(_tlayer(pre4, idx36b, params["tl4"], EFD), params["bn4"])

    idx12a, _ = _knn(xyz, xyz512, 12)
    f4_up = _indicator(params["i5"], f4, xyz512, xyz, idx12a)

    pre5 = jnp.concatenate([f4_up, xyz], axis=-1)
    f5 = _bn(_tlayer(pre5, idx36, params["tl5"], EFD), params["bn5"])

    idx12b, w = _knn(detect_point, xyz, 12)
    featc = jnp.concatenate([f3, f5], axis=-1)
    occ = _indicator(params["i3"], featc, xyz, detect_point, idx12b,
                     cls=params["cls"])
    return occ, w.reshape(w.shape[0], w.shape[1])


# speedup vs baseline: 2.6738x; 1.4801x over previous
import functools
import math

import jax
import jax.numpy as jnp
from jax import lax
from jax.experimental import pallas as pl
from jax.experimental.pallas import tpu as pltpu
from jax.experimental.pallas import tpu_sc as plsc

_HIGH = lax.Precision.HIGHEST
_INTERP = False

EFD = 32
_SC_RC = 128


def _dot(a, b):
    return lax.dot_general(a, b, (((1,), (0,)), ((), ())), precision=_HIGH)


def _dot_nt(a, b):
    return lax.dot_general(a, b, (((1,), (1,)), ((), ())), precision=_HIGH)


def _pad16(c):
    return (c + 15) // 16 * 16


def _pad128(c):
    return (c + 127) // 128 * 128


def _sc_gather(table, idx):
    v, d = table.shape
    r = idx.shape[0]
    rc = _SC_RC
    info = plsc.get_sparse_core_info()
    nw = info.num_cores * info.num_subcores
    nc_ = info.num_cores
    nch = r // (nw * rc)

    @functools.partial(
        pl.kernel,
        mesh=plsc.VectorSubcoreMesh(core_axis_name="c", subcore_axis_name="s"),
        out_type=jax.ShapeDtypeStruct((r, d), jnp.float32),
        scratch_types=[
            pltpu.VMEM((rc,), jnp.int32),
            pltpu.VMEM((rc,), jnp.int32),
            pltpu.VMEM((rc, d), jnp.float32),
            pltpu.VMEM((rc, d), jnp.float32),
            pltpu.SemaphoreType.DMA,
            pltpu.SemaphoreType.DMA,
        ],
    )
    def k(tab_hbm, idx_hbm, out_hbm, idx0, idx1, rows0, rows1, sem0, sem1):
        wid = lax.axis_index("s") * nc_ + lax.axis_index("c")
        base = wid * nch
        idxs = (idx0, idx1)
        bufs = (rows0, rows1)
        sems = (sem0, sem1)
        cps = [None, None]
        pltpu.sync_copy(idx_hbm.at[pl.ds(base * rc, rc)], idx0)
        cps[0] = pltpu.async_copy(tab_hbm.at[idx0], rows0, sem0)
        for c in range(nch):
            if c + 1 < nch:
                nb = (c + 1) % 2
                pltpu.sync_copy(idx_hbm.at[pl.ds((base + c + 1) * rc, rc)],
                                idxs[nb])
                cps[nb] = pltpu.async_copy(tab_hbm.at[idxs[nb]],
                                           bufs[nb], sems[nb])
            cb = c % 2
            cps[cb].wait()
            pltpu.sync_copy(bufs[cb], out_hbm.at[pl.ds((base + c) * rc, rc)])

    return k(table, idx)


def _gather_nbr(tab, idxg, k):
    bb, v, d = tab.shape
    q = idxg.shape[0] // (k * bb)
    flat = _sc_gather(tab.reshape(bb * v, d), idxg)
    return flat.reshape(k, bb, q, d)


def _knn_body(k, q_ref, c_ref, idx_ref, w_ref):
    q = q_ref[0]
    c = c_ref[0]
    qb = q.shape[0]
    cn = c.shape[0]
    qaug = jnp.concatenate([-2.0 * q, jnp.ones((qb, 1), jnp.float32)], axis=1)
    csq = jnp.sum(c * c, axis=1, keepdims=True)
    caug = jnp.concatenate([c, csq], axis=1)
    s = _dot_nt(qaug, caug) + jnp.sum(q * q, axis=1, keepdims=True)
    iota = lax.broadcasted_iota(jnp.int32, (qb, cn), 1)
    cols = []
    for j in range(k):
        m = jnp.min(s, axis=1, keepdims=True)
        if j == 0:
            w_ref[0] = jnp.where(m > 0.03, 10.0, 1.0)
        sel = jnp.where(s <= m, iota, cn)
        ij = jnp.min(sel, axis=1, keepdims=True)
        cols.append(ij)
        s = jnp.where(iota == ij, jnp.float32(3.0e38), s)
    idx_ref[0] = jnp.concatenate(cols, axis=1)


def _knn(query, cand, k, qblk=256):
    bb, qn, _ = query.shape
    cn = cand.shape[1]
    idx, w = pl.pallas_call(
        functools.partial(_knn_body, k),
        grid=(bb, qn // qblk),
        in_specs=[pl.BlockSpec((1, qblk, 3), lambda b, i: (b, i, 0)),
                  pl.BlockSpec((1, cn, 3), lambda b, i: (b, 0, 0))],
        out_specs=(pl.BlockSpec((1, qblk, k), lambda b, i: (b, i, 0)),
                   pl.BlockSpec((1, qblk, 1), lambda b, i: (b, i, 0))),
        out_shape=(jax.ShapeDtypeStruct((bb, qn, k), jnp.int32),
                   jax.ShapeDtypeStruct((bb, qn, 1), jnp.float32)),
        interpret=_INTERP,
    )(query, cand)
    return idx, w


def _fps_body(npoint, dp, xyz_ref, f3_ref, vw_ref, vb_ref,
              tab_ref, ox_ref, oh_ref):
    bb = xyz_ref.shape[0]
    n = xyz_ref.shape[1]
    riota = lax.broadcasted_iota(jnp.int32, (n, 1), 0)
    liota = lax.broadcasted_iota(jnp.int32, (1, n), 1)
    xyzs = [xyz_ref[b] for b in range(bb)]

    def body(i, carry):
        dists, fars = carry
        new_d, new_f = [], []
        for b in range(bb):
            oh_ref[pl.ds(b * npoint + i, 1), :] = (
                (liota == fars[b]).astype(jnp.float32))
            cen = xyz_ref[b, pl.ds(fars[b], 1), :]
            d = jnp.sum((xyzs[b] - cen) ** 2, axis=1, keepdims=True)
            dist = jnp.minimum(dists[b], d)
            m = jnp.max(dist)
            far2 = jnp.min(jnp.where(dist >= m, riota, n)).astype(jnp.int32)
            new_d.append(dist)
            new_f.append(far2)
        return tuple(new_d), tuple(new_f)

    dist0 = tuple(jnp.full((n, 1), 1e10, jnp.float32) for _ in range(bb))
    far0 = tuple(jnp.int32(0) for _ in range(bb))
    lax.fori_loop(0, npoint, body, (dist0, far0))
    cf = f3_ref.shape[-1]
    dout = vb_ref.shape[-1]
    dd = cf + 3
    for b in range(bb):
        ohb = oh_ref[pl.ds(b * npoint, npoint), :]
        f3s = _dot(ohb, f3_ref[b])
        x512 = _dot(ohb, xyzs[b])
        pre = jnp.concatenate([f3s, x512], axis=1)
        gv = jnp.maximum(_dot(pre, vw_ref[...]) + vb_ref[...], 0.0)
        pad = jnp.zeros((npoint, dp - dout - dd), jnp.float32)
        tab_ref[b] = jnp.concatenate([gv, pre, pad], axis=1)
        ox_ref[b] = x512


def _fps_tab(xyz, f3, vw, vb, npoint=512):
    bb, n, _ = xyz.shape
    cf = f3.shape[-1]
    dout = vw.shape[-1]
    dp = _pad128(dout + cf + 3)
    fullspec = lambda a: pl.BlockSpec(a.shape, lambda: (0,) * a.ndim)
    vb2 = vb.reshape(1, dout)
    tab, x512 = pl.pallas_call(
        functools.partial(_fps_body, npoint, dp),
        in_specs=[fullspec(xyz), fullspec(f3), fullspec(vw), fullspec(vb2)],
        out_specs=(pl.BlockSpec((bb, npoint, dp), lambda: (0, 0, 0)),
                   pl.BlockSpec((bb, npoint, 3), lambda: (0, 0, 0))),
        out_shape=(jax.ShapeDtypeStruct((bb, npoint, dp), jnp.float32),
                   jax.ShapeDtypeStruct((bb, npoint, 3), jnp.float32)),
        scratch_shapes=[pltpu.VMEM((bb * npoint, n), jnp.float32)],
        interpret=_INTERP,
    )(xyz, f3, vw, vb2)
    return tab, x512


def _tl_body(knn, jc, dout, dd, pb, tab_ref, nbr_ref, rw1_ref, rw2_ref,
             rb_ref, sw_ref, sb_ref, sum_ref, exp_ref, p_ref, out_ref):
    q = tab_ref[0][:, dout:dout + dd]
    rw1 = rw1_ref[...]
    rwc = rw2_ref[...] - rw1
    qcomb = _dot(q, rwc) + rb_ref[...]
    qrep = jnp.concatenate([qcomb] * jc, axis=0)
    smat = sum_ref[...]
    emat = exp_ref[...]
    pmat = p_ref[...]
    rootd = math.sqrt(dout)
    feat = jnp.zeros((pb, dout), jnp.float32)
    for t in range(knn // jc):
        nb = nbr_ref[pl.ds(t * jc, jc), 0, :, :]
        nbf = nb.reshape(jc * pb, nb.shape[-1])
        gvn = nbf[:, :dout]
        mj = _dot(nbf[:, dout:dout + dd], rw1) + qrep
        sj = _dot(jnp.abs(mj), smat) + jnp.float32(dout * 1e-7)
        a = gvn * rootd / sj
        a_exp = _dot(a, emat)
        contrib = _dot(a_exp * mj, pmat)
        feat = feat + jnp.sum(contrib.reshape(jc, pb, dout), axis=0)
    out_ref[0] = _dot(feat, sw_ref[...]) + sb_ref[...]


def _tlayer(tab, nbr, p, dout, dd, pblk=256, jc=6):
    bb, n, dp = tab.shape
    knn = nbr.shape[0]
    d2 = dout * dout
    rw1 = p["r_W"][:dd]
    rw2 = p["r_W"][dd:]
    rb = p["r_b"].reshape(1, d2)
    sw = p["s_W"]
    sb = p["s_b"].reshape(1, dout)
    eye = jnp.eye(dout, dtype=jnp.float32)
    ones = jnp.ones((dout, 1), jnp.float32)
    smat = jnp.kron(eye, ones)
    pmat = jnp.kron(ones, eye)
    emat = smat.T
    wspec = lambda w: pl.BlockSpec(w.shape, lambda b, i: (0,) * w.ndim)
    out = pl.pallas_call(
        functools.partial(_tl_body, knn, jc, dout, dd, pblk),
        grid=(bb, n // pblk),
        in_specs=[pl.BlockSpec((1, pblk, dp), lambda b, i: (b, i, 0)),
                  pl.BlockSpec((knn, 1, pblk, dp), lambda b, i: (0, b, i, 0)),
                  wspec(rw1), wspec(rw2), wspec(rb),
                  wspec(sw), wspec(sb), wspec(smat), wspec(emat),
                  wspec(pmat)],
        out_specs=pl.BlockSpec((1, pblk, dout), lambda b, i: (b, i, 0)),
        out_shape=jax.ShapeDtypeStruct((bb, n, dout), jnp.float32),
        interpret=_INTERP,
    )(tab, nbr, rw1, rw2, rb, sw, sb, smat, emat, pmat)
    return out


def _bn_body(x_ref, g_ref, b_ref, o_ref):
    x = x_ref[...]
    m = jnp.mean(x, axis=(0, 1), keepdims=True)
    v = jnp.mean((x - m) ** 2, axis=(0, 1), keepdims=True)
    o_ref[...] = (x - m) / jnp.sqrt(v + 1e-5) * g_ref[...] + b_ref[...]


def _bn(x, p):
    bb, n, c = x.shape
    g = p["gamma"].reshape(1, 1, c)
    b = p["beta"].reshape(1, 1, c)
    fullspec = lambda a: pl.BlockSpec(a.shape, lambda: (0,) * a.ndim)
    return pl.pallas_call(
        _bn_body,
        in_specs=[fullspec(x), fullspec(g), fullspec(b)],
        out_specs=pl.BlockSpec(x.shape, lambda: (0, 0, 0)),
        out_shape=jax.ShapeDtypeStruct(x.shape, jnp.float32),
        interpret=_INTERP,
    )(x, g, b)


def _bn_tab_body(mode, dp, x_ref, g_ref, b_ref, xyz_ref, ex_ref, w_ref,
                 wb_ref, o_ref):
    x = x_ref[...]
    bb, n, c = x.shape
    m = jnp.mean(x, axis=(0, 1), keepdims=True)
    v = jnp.mean((x - m) ** 2, axis=(0, 1), keepdims=True)
    xb = (x - m) / jnp.sqrt(v + 1e-5) * g_ref[...] + b_ref[...]
    xyz = xyz_ref[...]
    dout = wb_ref.shape[-1]
    for b in range(bb):
        if mode == "tl":
            pre = jnp.concatenate([xb[b], xyz[b]], axis=1)
            lin = jnp.maximum(_dot(pre, w_ref[...]) + wb_ref[...], 0.0)
            pay = pre
        else:
            lin_in = xb[b] if ex_ref is None else (
                jnp.concatenate([ex_ref[b], xb[b]], axis=1))
            lin = _dot(lin_in, w_ref[...]) + wb_ref[...]
            pay = xyz[b]
        pad = jnp.zeros((n, dp - dout - pay.shape[-1]), jnp.float32)
        o_ref[b] = jnp.concatenate([lin, pay, pad], axis=1)


def _bn_tab(x, p, xyz, w, wb, mode, extra=None):
    bb, n, c = x.shape
    dout = w.shape[-1]
    pay = (c + 3) if mode == "tl" else 3
    dp = _pad128(dout + pay)
    g = p["gamma"].reshape(1, 1, c)
    b = p["beta"].reshape(1, 1, c)
    wb2 = wb.reshape(1, dout)
    fullspec = lambda a: pl.BlockSpec(a.shape, lambda: (0,) * a.ndim)
    args = [x, g, b, xyz]
    specs = [fullspec(x), fullspec(g), fullspec(b), fullspec(xyz)]
    if extra is not None:
        args.append(extra)
        specs.append(fullspec(extra))
    args += [w, wb2]
    specs += [fullspec(w), fullspec(wb2)]

    def body(*refs):
        if extra is not None:
            x_r, g_r, b_r, xyz_r, ex_r, w_r, wb_r, o_r = refs
        else:
            x_r, g_r, b_r, xyz_r, w_r, wb_r, o_r = refs
            ex_r = None
        _bn_tab_body(mode, dp, x_r, g_r, b_r, xyz_r, ex_r, w_r, wb_r, o_r)

    return pl.pallas_call(
        body,
        in_specs=specs,
        out_specs=pl.BlockSpec((bb, n, dp), lambda: (0, 0, 0)),
        out_shape=jax.ShapeDtypeStruct((bb, n, dp), jnp.float32),
        interpret=_INTERP,
    )(*args)


def _tab1_body(dp, dout, xyz_ref, vw_ref, vb_ref, o_ref):
    x = xyz_ref[0]
    n = x.shape[0]
    pre = jnp.concatenate([x, x], axis=1)
    gv = jnp.maximum(_dot(pre, vw_ref[...]) + vb_ref[...], 0.0)
    pad = jnp.zeros((n, dp - dout - 6), jnp.float32)
    o_ref[0] = jnp.concatenate([gv, pre, pad], axis=1)


def _tab1(xyz, vw, vb):
    bb, n, _ = xyz.shape
    dout = vw.shape[-1]
    dp = _pad128(dout + 6)
    vb2 = vb.reshape(1, dout)
    wspec = lambda w: pl.BlockSpec(w.shape, lambda b: (0,) * w.ndim)
    return pl.pallas_call(
        functools.partial(_tab1_body, dp, dout),
        grid=(bb,),
        in_specs=[pl.BlockSpec((1, n, 3), lambda b: (b, 0, 0)),
                  wspec(vw), wspec(vb2)],
        out_specs=pl.BlockSpec((1, n, dp), lambda b: (b, 0, 0)),
        out_shape=jax.ShapeDtypeStruct((bb, n, dp), jnp.float32),
        interpret=_INTERP,
    )(xyz, vw, vb2)


def _ind_body(knn, dout, dd, pb, tailmode, nbr_ref, x2_ref, p1w_ref, p1b_ref,
              p2w_ref, p2b_ref, a_ref, ab_ref, b_ref, bb_ref, out_ref):
    q = x2_ref[0]
    qrep = jnp.concatenate([q] * knn, axis=0)
    nb = nbr_ref[:, 0, :, :]
    nbf = nb.reshape(knn * pb, nb.shape[-1])
    ftn = nbf[:, :dout]
    gx = nbf[:, dout:dout + 3] - qrep
    h = jnp.maximum(_dot(gx, p1w_ref[...]) + p1b_ref[...], 0.0)
    pw = _dot(h, p2w_ref[...]) + p2b_ref[...]
    c = pw * ftn
    nf = jnp.sum(c.reshape(knn, pb, dout), axis=0) / math.sqrt(knn)
    if tailmode == "cls":
        h2 = jnp.maximum(_dot(nf, a_ref[...]) + ab_ref[...], 0.0)
        out_ref[0] = _dot(h2, b_ref[...]) + bb_ref[...]
    else:
        pre = jnp.concatenate([nf, q], axis=1)
        gv = jnp.maximum(_dot(pre, a_ref[...]) + ab_ref[...], 0.0)
        gd = ab_ref.shape[-1]
        pad = jnp.zeros((pb, out_ref.shape[-1] - gd - dout - 3), jnp.float32)
        out_ref[0] = jnp.concatenate([gv, pre, pad], axis=1)


def _indicator(p, nbr, x2, tailmode, wa, wab, wb_, wbb, pblk=256):
    knn, bb, qn, dp = nbr.shape
    dout = p["p1_W"].shape[1]
    p1w = p["p1_W"]
    p1b = p["p1_b"].reshape(1, dout)
    p2w = p["p2_W"]
    p2b = p["p2_b"].reshape(1, dout)
    wab2 = wab.reshape(1, -1)
    wbb2 = wbb.reshape(1, -1)
    if tailmode == "cls":
        cout = wb_.shape[-1]
    else:
        cout = _pad128(wa.shape[-1] + dout + 3)
    wspec = lambda w: pl.BlockSpec(w.shape, lambda b, i: (0,) * w.ndim)
    out = pl.pallas_call(
        functools.partial(_ind_body, knn, dout, dp, pblk, tailmode),
        grid=(bb, qn // pblk),
        in_specs=[pl.BlockSpec((knn, 1, pblk, dp), lambda b, i: (0, b, i, 0)),
                  pl.BlockSpec((1, pblk, 3), lambda b, i: (b, i, 0)),
                  wspec(p1w), wspec(p1b), wspec(p2w), wspec(p2b),
                  wspec(wa), wspec(wab2), wspec(wb_), wspec(wbb2)],
        out_specs=pl.BlockSpec((1, pblk, cout), lambda b, i: (b, i, 0)),
        out_shape=jax.ShapeDtypeStruct((bb, qn, cout), jnp.float32),
        interpret=_INTERP,
    )(nbr, x2, p1w, p1b, p2w, p2b, wa, wab2, wb_, wbb2)
    return out


def _flat_idx(idx, v):
    bb = idx.shape[0]
    off = (jnp.arange(bb, dtype=jnp.int32) * v)[:, None, None]
    return jnp.transpose(idx + off, (2, 0, 1)).reshape(-1)


def kernel(xyz, detect_point, normal_gt, params):
    del normal_gt
    bb, n, _ = xyz.shape
    tl1, tl2, tl3 = params["tl1"], params["tl2"], params["tl3"]
    tl4, tl5 = params["tl4"], params["tl5"]
    i5p, i3p, cls = params["i5"], params["i3"], params["cls"]

    idx36, _ = _knn(xyz, xyz, 36)
    g36 = _flat_idx(idx36, n)

    tab1 = _tab1(xyz, tl1["v_W"], tl1["v_b"])
    nbr1 = _gather_nbr(tab1, g36, 36)
    f1 = _tlayer(tab1, nbr1, tl1, EFD // 4, 6)

    tab2 = _bn_tab(f1, params["bn1"], xyz, tl2["v_W"], tl2["v_b"], "tl")
    nbr2 = _gather_nbr(tab2, g36, 36)
    f2 = _tlayer(tab2, nbr2, tl2, EFD, 11)

    tab3 = _bn_tab(f2, params["bn2"], xyz, tl3["v_W"], tl3["v_b"], "tl")
    nbr3 = _gather_nbr(tab3, g36, 36)
    f3 = _tlayer(tab3, nbr3, tl3, EFD, 35)
    f3b = _bn(f3, params["bn3"])

    tab4, xyz512 = _fps_tab(xyz, f3b, tl4["v_W"], tl4["v_b"])
    idx36b, _ = _knn(xyz512, xyz512, 36)
    g36b = _flat_idx(idx36b, 512)
    nbr4 = _gather_nbr(tab4, g36b, 36)
    f4 = _tlayer(tab4, nbr4, tl4, EFD, 35)

    tab5 = _bn_tab(f4, params["bn4"], xyz512, i5p["pre_W"], i5p["pre_b"],
                   "ind")
    idx12a, _ = _knn(xyz, xyz512, 12)
    g12a = _flat_idx(idx12a, 512)
    nbr5 = _gather_nbr(tab5, g12a, 12)
    tab6 = _indicator(i5p, nbr5, xyz, "tab",
                      tl5["v_W"], tl5["v_b"], tl5["v_W"], tl5["v_b"])

    nbr6 = _gather_nbr(tab6, g36, 36)
    f5 = _tlayer(tab6, nbr6, tl5, EFD, 35)

    tab7 = _bn_tab(f5, params["bn5"], xyz, i3p["pre_W"], i3p["pre_b"],
                   "ind", extra=f3b)
    idx12b, w = _knn(detect_point, xyz, 12)
    g12b = _flat_idx(idx12b, n)
    nbr7 = _gather_nbr(tab7, g12b, 12)
    occ = _indicator(i3p, nbr7, detect_point, "cls",
                     cls["c1_W"], cls["c1_b"], cls["c2_W"], cls["c2_b"])
    return occ, w.reshape(w.shape[0], w.shape[1])
